# Initial kernel scaffold; baseline (speedup 1.0000x reference)
import jax
import jax.numpy as jnp
from jax.experimental import pallas as pl


def kernel(q_feat, kv_feat, pos_q, pos_k, heading_q, heading_k, mask_k, Wq, bq, Wk, bk, Wv, bv, Wo, bo):
    raise NotImplementedError("write your pallas kernel here")


# speedup vs baseline: 15.6930x; 15.6930x over previous
import functools

import jax
import jax.numpy as jnp
from jax.experimental import pallas as pl

_NEG = -1e30
_BIG = 1e30


def _rot_tables(pos, heading, quarter):
    idx = jax.lax.broadcasted_iota(jnp.int32, (1, quarter), 1).astype(jnp.float32)
    freqs = jnp.exp(idx * (-jnp.log(10000.0) / quarter))
    ax = pos[:, 0:1] * freqs
    ay = pos[:, 1:2] * freqs
    rc = jnp.concatenate([jnp.cos(ax), jnp.cos(ay)], axis=1)
    rs = jnp.concatenate([jnp.sin(ax), jnp.sin(ay)], axis=1)
    hc = jnp.cos(heading)
    hs = jnp.sin(heading)
    return rc, rs, hc, hs


def _encode_heads(P, rc, rs, hc, hs, H, DH):
    half = DH // 2
    heads = []
    for h in range(H):
        p1 = P[:, h * DH:h * DH + half]
        p2 = P[:, h * DH + half:(h + 1) * DH]
        c, s = (rc, rs) if h % 2 == 0 else (hc, hs)
        heads.append(jnp.concatenate([p1 * c - p2 * s, p2 * c + p1 * s], axis=1))
    return heads


def _proj_encode_kernel(qf, kvf, posq, posk, hq, hk,
                        wq, bq, wk, bk, wv, bv,
                        qe, ke, vv, *, H, DH):
    dn = (((1,), (1,)), ((), ()))
    quarter = DH // 4
    Pq = jax.lax.dot_general(qf[0], wq[...], dn,
                             preferred_element_type=jnp.float32) + bq[...]
    Pk = jax.lax.dot_general(kvf[0], wk[...], dn,
                             preferred_element_type=jnp.float32) + bk[...]
    Pv = jax.lax.dot_general(kvf[0], wv[...], dn,
                             preferred_element_type=jnp.float32) + bv[...]

    rc, rs, hc, hs = _rot_tables(posq[0], hq[0], quarter)
    for h, piece in enumerate(_encode_heads(Pq, rc, rs, hc, hs, H, DH)):
        qe[0, h] = piece
    rc, rs, hc, hs = _rot_tables(posk[0], hk[0], quarter)
    for h, piece in enumerate(_encode_heads(Pk, rc, rs, hc, hs, H, DH)):
        ke[0, h] = piece
    for h in range(H):
        vv[0, h] = Pv[:, h * DH:(h + 1) * DH]


def _attn_kernel(qe, ke, vv, posq, poskT, wo, bo, out, *, H, DH, K, NK):
    BQ = out.shape[1]
    scale = 1.0 / (DH ** 0.5)

    xq = posq[0][:, 0:1]
    yq = posq[0][:, 1:2]
    xk = poskT[0][0:1, :]
    yk = poskT[0][1:2, :]
    dist = jnp.sqrt((xq - xk) ** 2 + (yq - yk) ** 2)

    lane = jax.lax.broadcasted_iota(jnp.int32, (BQ, NK), 1)

    def body(_, carry):
        d, sel = carry
        m = jnp.min(d, axis=1, keepdims=True)
        first = jnp.min(jnp.where(d == m, lane, NK), axis=1, keepdims=True)
        kill = lane == first
        return jnp.where(kill, _BIG, d), jnp.where(kill, 1.0, sel)

    _, sel = jax.lax.fori_loop(
        0, K, body, (dist, jnp.zeros((BQ, NK), jnp.float32)))
    keep = sel > 0.0

    dn = (((1,), (1,)), ((), ()))
    outs = []
    for h in range(H):
        lg = jax.lax.dot_general(qe[0, h], ke[0, h], dn,
                                 preferred_element_type=jnp.float32) * scale
        lg = jnp.where(keep, lg, _NEG)
        m = jnp.max(lg, axis=1, keepdims=True)
        p = jnp.exp(lg - m)
        a = p / jnp.sum(p, axis=1, keepdims=True)
        outs.append(jnp.dot(a, vv[0, h], preferred_element_type=jnp.float32))
    O = jnp.concatenate(outs, axis=1)
    out[0] = jax.lax.dot_general(O, wo[...], dn,
                                 preferred_element_type=jnp.float32) + bo[...]


def kernel(q_feat, kv_feat, pos_q, pos_k, heading_q, heading_k, mask_k,
           Wq, bq, Wk, bk, Wv, bv, Wo, bo):
    B, NQ, D = q_feat.shape
    NK = kv_feat.shape[1]
    H = 8
    DH = D // H
    K = 32
    BQ = 256
    nq_blocks = NQ // BQ

    posq3 = pos_q
    posk3 = pos_k
    poskT = jnp.swapaxes(pos_k, 1, 2)
    hq3 = heading_q[..., None]
    hk3 = heading_k[..., None]
    bq2 = bq[None, :]
    bk2 = bk[None, :]
    bv2 = bv[None, :]
    bo2 = bo[None, :]

    enc_shape = jax.ShapeDtypeStruct((B, H, NQ, DH), jnp.float32)
    qe, ke, vv = pl.pallas_call(
        functools.partial(_proj_encode_kernel, H=H, DH=DH),
        grid=(B, nq_blocks),
        in_specs=[
            pl.BlockSpec((1, BQ, D), lambda b, i: (b, i, 0)),
            pl.BlockSpec((1, BQ, D), lambda b, i: (b, i, 0)),
            pl.BlockSpec((1, BQ, 2), lambda b, i: (b, i, 0)),
            pl.BlockSpec((1, BQ, 2), lambda b, i: (b, i, 0)),
            pl.BlockSpec((1, BQ, 1), lambda b, i: (b, i, 0)),
            pl.BlockSpec((1, BQ, 1), lambda b, i: (b, i, 0)),
            pl.BlockSpec((D, D), lambda b, i: (0, 0)),
            pl.BlockSpec((1, D), lambda b, i: (0, 0)),
            pl.BlockSpec((D, D), lambda b, i: (0, 0)),
            pl.BlockSpec((1, D), lambda b, i: (0, 0)),
            pl.BlockSpec((D, D), lambda b, i: (0, 0)),
            pl.BlockSpec((1, D), lambda b, i: (0, 0)),
        ],
        out_specs=[
            pl.BlockSpec((1, H, BQ, DH), lambda b, i: (b, 0, i, 0)),
            pl.BlockSpec((1, H, BQ, DH), lambda b, i: (b, 0, i, 0)),
            pl.BlockSpec((1, H, BQ, DH), lambda b, i: (b, 0, i, 0)),
        ],
        out_shape=[enc_shape, enc_shape, enc_shape],
    )(q_feat, kv_feat, posq3, posk3, hq3, hk3, Wq, bq2, Wk, bk2, Wv, bv2)

    out = pl.pallas_call(
        functools.partial(_attn_kernel, H=H, DH=DH, K=K, NK=NK),
        grid=(B, nq_blocks),
        in_specs=[
            pl.BlockSpec((1, H, BQ, DH), lambda b, i: (b, 0, i, 0)),
            pl.BlockSpec((1, H, NK, DH), lambda b, i: (b, 0, 0, 0)),
            pl.BlockSpec((1, H, NK, DH), lambda b, i: (b, 0, 0, 0)),
            pl.BlockSpec((1, BQ, 2), lambda b, i: (b, i, 0)),
            pl.BlockSpec((1, 2, NK), lambda b, i: (b, 0, 0)),
            pl.BlockSpec((D, D), lambda b, i: (0, 0)),
            pl.BlockSpec((1, D), lambda b, i: (0, 0)),
        ],
        out_specs=pl.BlockSpec((1, BQ, D), lambda b, i: (b, i, 0)),
        out_shape=jax.ShapeDtypeStruct((B, NQ, D), jnp.float32),
    )(qe, ke, vv, posq3, poskT, Wo, bo2)

    return out


# speedup vs baseline: 30.9633x; 1.9731x over previous
import functools

import jax
import jax.numpy as jnp
from jax.experimental import pallas as pl
from jax.experimental.pallas import tpu as pltpu


def _rot_tables(pos, heading, quarter):
    idx = jax.lax.broadcasted_iota(jnp.int32, (1, quarter), 1).astype(jnp.float32)
    freqs = jnp.exp(idx * (-jnp.log(10000.0) / quarter))
    ax = pos[:, 0:1] * freqs
    ay = pos[:, 1:2] * freqs
    rc = jnp.concatenate([jnp.cos(ax), jnp.cos(ay)], axis=1)
    rs = jnp.concatenate([jnp.sin(ax), jnp.sin(ay)], axis=1)
    hc = jnp.cos(heading)
    hs = jnp.sin(heading)
    return rc, rs, hc, hs


def _encode_heads(P, rc, rs, hc, hs, H, DH, gain):
    half = DH // 2
    heads = []
    for h in range(H):
        p1 = P[:, h * DH:h * DH + half]
        p2 = P[:, h * DH + half:(h + 1) * DH]
        c, s = (rc, rs) if h % 2 == 0 else (hc, hs)
        heads.append(jnp.concatenate(
            [(p1 * c - p2 * s) * gain, (p2 * c + p1 * s) * gain], axis=1))
    return heads


def _proj_encode_kernel(qf, kvf, posq, posk, hq, hk,
                        wq, bq, wk, bk, wv, bv,
                        qe, ke, vv, *, H, DH):
    dn = (((1,), (1,)), ((), ()))
    quarter = DH // 4
    scale = 1.0 / (DH ** 0.5)
    Pq = jax.lax.dot_general(qf[0], wq[...], dn,
                             preferred_element_type=jnp.float32) + bq[...]
    Pk = jax.lax.dot_general(kvf[0], wk[...], dn,
                             preferred_element_type=jnp.float32) + bk[...]
    Pv = jax.lax.dot_general(kvf[0], wv[...], dn,
                             preferred_element_type=jnp.float32) + bv[...]

    rc, rs, hc, hs = _rot_tables(posq[0], hq[0], quarter)
    for h, piece in enumerate(_encode_heads(Pq, rc, rs, hc, hs, H, DH, scale)):
        qe[0, h] = piece
    rc, rs, hc, hs = _rot_tables(posk[0], hk[0], quarter)
    for h, piece in enumerate(_encode_heads(Pk, rc, rs, hc, hs, H, DH, 1.0)):
        ke[0, h] = piece
    for h in range(H):
        vv[0, h] = Pv[:, h * DH:(h + 1) * DH]


def _attn_kernel(qe, ke, vv, posq, poskT, wo, bo, out, *, H, DH, K, NK):
    BQ = out.shape[1]

    xq = posq[0][:, 0:1]
    yq = posq[0][:, 1:2]
    xk = poskT[0][0:1, :]
    yk = poskT[0][1:2, :]
    d2 = (xq - xk) ** 2 + (yq - yk) ** 2

    r = jax.lax.bitcast_convert_type(d2, jnp.int32)

    def bit_step(i, v):
        cand = v | (jnp.int32(1) << (jnp.int32(30) - i))
        cnt = jnp.sum(jnp.where(r < cand, jnp.int32(1), jnp.int32(0)),
                      axis=1, keepdims=True)
        return jnp.where(cnt <= K - 1, cand, v)

    v32 = jax.lax.fori_loop(0, 31, bit_step,
                            jnp.zeros((BQ, 1), jnp.int32))
    keep = r <= v32

    dn = (((1,), (1,)), ((), ()))
    outs = []
    for h in range(H):
        lg = jax.lax.dot_general(qe[0, h], ke[0, h], dn,
                                 preferred_element_type=jnp.float32)
        p = jnp.where(keep, jnp.exp(lg), 0.0)
        inv = 1.0 / jnp.sum(p, axis=1, keepdims=True)
        outs.append(jnp.dot(p, vv[0, h],
                            preferred_element_type=jnp.float32) * inv)
    O = jnp.concatenate(outs, axis=1)
    out[0] = jax.lax.dot_general(O, wo[...], dn,
                                 preferred_element_type=jnp.float32) + bo[...]


def kernel(q_feat, kv_feat, pos_q, pos_k, heading_q, heading_k, mask_k,
           Wq, bq, Wk, bk, Wv, bv, Wo, bo):
    B, NQ, D = q_feat.shape
    NK = kv_feat.shape[1]
    H = 8
    DH = D // H
    K = 32
    BQ = 256
    nq_blocks = NQ // BQ

    poskT = jnp.swapaxes(pos_k, 1, 2)
    hq3 = heading_q[..., None]
    hk3 = heading_k[..., None]
    bq2 = bq[None, :]
    bk2 = bk[None, :]
    bv2 = bv[None, :]
    bo2 = bo[None, :]

    enc_shape = jax.ShapeDtypeStruct((B, H, NQ, DH), jnp.float32)
    qe, ke, vv = pl.pallas_call(
        functools.partial(_proj_encode_kernel, H=H, DH=DH),
        grid=(B, nq_blocks),
        in_specs=[
            pl.BlockSpec((1, BQ, D), lambda b, i: (b, i, 0)),
            pl.BlockSpec((1, BQ, D), lambda b, i: (b, i, 0)),
            pl.BlockSpec((1, BQ, 2), lambda b, i: (b, i, 0)),
            pl.BlockSpec((1, BQ, 2), lambda b, i: (b, i, 0)),
            pl.BlockSpec((1, BQ, 1), lambda b, i: (b, i, 0)),
            pl.BlockSpec((1, BQ, 1), lambda b, i: (b, i, 0)),
            pl.BlockSpec((D, D), lambda b, i: (0, 0)),
            pl.BlockSpec((1, D), lambda b, i: (0, 0)),
            pl.BlockSpec((D, D), lambda b, i: (0, 0)),
            pl.BlockSpec((1, D), lambda b, i: (0, 0)),
            pl.BlockSpec((D, D), lambda b, i: (0, 0)),
            pl.BlockSpec((1, D), lambda b, i: (0, 0)),
        ],
        out_specs=[
            pl.BlockSpec((1, H, BQ, DH), lambda b, i: (b, 0, i, 0)),
            pl.BlockSpec((1, H, BQ, DH), lambda b, i: (b, 0, i, 0)),
            pl.BlockSpec((1, H, BQ, DH), lambda b, i: (b, 0, i, 0)),
        ],
        out_shape=[enc_shape, enc_shape, enc_shape],
        compiler_params=pltpu.CompilerParams(
            dimension_semantics=("parallel", "parallel")),
    )(q_feat, kv_feat, pos_q, pos_k, hq3, hk3, Wq, bq2, Wk, bk2, Wv, bv2)

    out = pl.pallas_call(
        functools.partial(_attn_kernel, H=H, DH=DH, K=K, NK=NK),
        grid=(B, nq_blocks),
        in_specs=[
            pl.BlockSpec((1, H, BQ, DH), lambda b, i: (b, 0, i, 0)),
            pl.BlockSpec((1, H, NK, DH), lambda b, i: (b, 0, 0, 0)),
            pl.BlockSpec((1, H, NK, DH), lambda b, i: (b, 0, 0, 0)),
            pl.BlockSpec((1, BQ, 2), lambda b, i: (b, i, 0)),
            pl.BlockSpec((1, 2, NK), lambda b, i: (b, 0, 0)),
            pl.BlockSpec((D, D), lambda b, i: (0, 0)),
            pl.BlockSpec((1, D), lambda b, i: (0, 0)),
        ],
        out_specs=pl.BlockSpec((1, BQ, D), lambda b, i: (b, i, 0)),
        out_shape=jax.ShapeDtypeStruct((B, NQ, D), jnp.float32),
        compiler_params=pltpu.CompilerParams(
            dimension_semantics=("parallel", "parallel")),
    )(qe, ke, vv, pos_q, poskT, Wo, bo2)

    return out


# speedup vs baseline: 33.9379x; 1.0961x over previous
import functools

import jax
import jax.numpy as jnp
from jax.experimental import pallas as pl
from jax.experimental.pallas import tpu as pltpu


def _rot_tables(pos, heading, quarter):
    idx = jax.lax.broadcasted_iota(jnp.int32, (1, quarter), 1).astype(jnp.float32)
    freqs = jnp.exp(idx * (-jnp.log(10000.0) / quarter))
    ax = pos[:, 0:1] * freqs
    ay = pos[:, 1:2] * freqs
    rc = jnp.concatenate([jnp.cos(ax), jnp.cos(ay)], axis=1)
    rs = jnp.concatenate([jnp.sin(ax), jnp.sin(ay)], axis=1)
    hc = jnp.cos(heading)
    hs = jnp.sin(heading)
    return rc, rs, hc, hs


def _encode_heads(P, rc, rs, hc, hs, H, DH, gain):
    half = DH // 2
    heads = []
    for h in range(H):
        p1 = P[:, h * DH:h * DH + half]
        p2 = P[:, h * DH + half:(h + 1) * DH]
        c, s = (rc, rs) if h % 2 == 0 else (hc, hs)
        heads.append(jnp.concatenate(
            [(p1 * c - p2 * s) * gain, (p2 * c + p1 * s) * gain], axis=1))
    return heads


def _proj_encode_kernel(qf, kvf, posq, posk, hq, hk,
                        wq, bq, wk, bk, wv, bv,
                        qe, ke, vv, *, H, DH):
    dn = (((1,), (1,)), ((), ()))
    quarter = DH // 4
    scale = 1.0 / (DH ** 0.5)
    Pq = jax.lax.dot_general(qf[0], wq[...], dn,
                             preferred_element_type=jnp.float32) + bq[...]
    Pk = jax.lax.dot_general(kvf[0], wk[...], dn,
                             preferred_element_type=jnp.float32) + bk[...]
    Pv = jax.lax.dot_general(kvf[0], wv[...], dn,
                             preferred_element_type=jnp.float32) + bv[...]

    rc, rs, hc, hs = _rot_tables(posq[0], hq[0], quarter)
    for h, piece in enumerate(_encode_heads(Pq, rc, rs, hc, hs, H, DH, scale)):
        qe[0, h] = piece
    rc, rs, hc, hs = _rot_tables(posk[0], hk[0], quarter)
    for h, piece in enumerate(_encode_heads(Pk, rc, rs, hc, hs, H, DH, 1.0)):
        ke[0, h] = piece
    for h in range(H):
        vv[0, h] = Pv[:, h * DH:(h + 1) * DH]


def _attn_kernel(qe, ke, vv, posq, poskT, wo, bo, out, *, H, DH, K, NK):
    BQ = out.shape[1]

    xq = posq[0][:, 0:1]
    yq = posq[0][:, 1:2]
    xk = poskT[0][0:1, :]
    yk = poskT[0][1:2, :]
    d2 = (xq - xk) ** 2 + (yq - yk) ** 2

    r = jax.lax.bitcast_convert_type(d2, jnp.int32)

    v32 = jnp.zeros((BQ, 1), jnp.int32)
    for bit in range(30, -1, -1):
        cand = v32 | jnp.int32(1 << bit)
        cnt = jnp.sum(jnp.where(r < cand, jnp.int32(1), jnp.int32(0)),
                      axis=1, keepdims=True)
        v32 = jnp.where(cnt <= K - 1, cand, v32)
    keep = r <= v32

    dn = (((1,), (1,)), ((), ()))
    outs = []
    for h in range(H):
        lg = jax.lax.dot_general(qe[0, h], ke[0, h], dn,
                                 preferred_element_type=jnp.float32)
        p = jnp.where(keep, jnp.exp(lg), 0.0)
        inv = 1.0 / jnp.sum(p, axis=1, keepdims=True)
        outs.append(jnp.dot(p, vv[0, h],
                            preferred_element_type=jnp.float32) * inv)
    O = jnp.concatenate(outs, axis=1)
    out[0] = jax.lax.dot_general(O, wo[...], dn,
                                 preferred_element_type=jnp.float32) + bo[...]


def kernel(q_feat, kv_feat, pos_q, pos_k, heading_q, heading_k, mask_k,
           Wq, bq, Wk, bk, Wv, bv, Wo, bo):
    B, NQ, D = q_feat.shape
    NK = kv_feat.shape[1]
    H = 8
    DH = D // H
    K = 32
    BQ = 256
    nq_blocks = NQ // BQ

    poskT = jnp.swapaxes(pos_k, 1, 2)
    hq3 = heading_q[..., None]
    hk3 = heading_k[..., None]
    bq2 = bq[None, :]
    bk2 = bk[None, :]
    bv2 = bv[None, :]
    bo2 = bo[None, :]

    enc_shape = jax.ShapeDtypeStruct((B, H, NQ, DH), jnp.float32)
    qe, ke, vv = pl.pallas_call(
        functools.partial(_proj_encode_kernel, H=H, DH=DH),
        grid=(B, nq_blocks),
        in_specs=[
            pl.BlockSpec((1, BQ, D), lambda b, i: (b, i, 0)),
            pl.BlockSpec((1, BQ, D), lambda b, i: (b, i, 0)),
            pl.BlockSpec((1, BQ, 2), lambda b, i: (b, i, 0)),
            pl.BlockSpec((1, BQ, 2), lambda b, i: (b, i, 0)),
            pl.BlockSpec((1, BQ, 1), lambda b, i: (b, i, 0)),
            pl.BlockSpec((1, BQ, 1), lambda b, i: (b, i, 0)),
            pl.BlockSpec((D, D), lambda b, i: (0, 0)),
            pl.BlockSpec((1, D), lambda b, i: (0, 0)),
            pl.BlockSpec((D, D), lambda b, i: (0, 0)),
            pl.BlockSpec((1, D), lambda b, i: (0, 0)),
            pl.BlockSpec((D, D), lambda b, i: (0, 0)),
            pl.BlockSpec((1, D), lambda b, i: (0, 0)),
        ],
        out_specs=[
            pl.BlockSpec((1, H, BQ, DH), lambda b, i: (b, 0, i, 0)),
            pl.BlockSpec((1, H, BQ, DH), lambda b, i: (b, 0, i, 0)),
            pl.BlockSpec((1, H, BQ, DH), lambda b, i: (b, 0, i, 0)),
        ],
        out_shape=[enc_shape, enc_shape, enc_shape],
        compiler_params=pltpu.CompilerParams(
            dimension_semantics=("parallel", "parallel")),
    )(q_feat, kv_feat, pos_q, pos_k, hq3, hk3, Wq, bq2, Wk, bk2, Wv, bv2)

    out = pl.pallas_call(
        functools.partial(_attn_kernel, H=H, DH=DH, K=K, NK=NK),
        grid=(B, nq_blocks),
        in_specs=[
            pl.BlockSpec((1, H, BQ, DH), lambda b, i: (b, 0, i, 0)),
            pl.BlockSpec((1, H, NK, DH), lambda b, i: (b, 0, 0, 0)),
            pl.BlockSpec((1, H, NK, DH), lambda b, i: (b, 0, 0, 0)),
            pl.BlockSpec((1, BQ, 2), lambda b, i: (b, i, 0)),
            pl.BlockSpec((1, 2, NK), lambda b, i: (b, 0, 0)),
            pl.BlockSpec((D, D), lambda b, i: (0, 0)),
            pl.BlockSpec((1, D), lambda b, i: (0, 0)),
        ],
        out_specs=pl.BlockSpec((1, BQ, D), lambda b, i: (b, i, 0)),
        out_shape=jax.ShapeDtypeStruct((B, NQ, D), jnp.float32),
        compiler_params=pltpu.CompilerParams(
            dimension_semantics=("parallel", "parallel")),
    )(qe, ke, vv, pos_q, poskT, Wo, bo2)

    return out


# speedup vs baseline: 44.7595x; 1.3189x over previous
import functools

import jax
import jax.numpy as jnp
from jax.experimental import pallas as pl
from jax.experimental.pallas import tpu as pltpu


def _rot_tables_qk(posq, posk, hq, hk, quarter):
    lane = jax.lax.broadcasted_iota(jnp.int32, (1, 16 * quarter), 1)
    sec = lane >> 4
    k16 = (lane & 15).astype(jnp.float32)
    fr = jnp.exp(k16 * (-jnp.log(10000.0) / quarter))
    axis_y = (sec & 1) == 1
    rope_v = jnp.where(sec < 4,
                       jnp.where(axis_y, posq[:, 1:2], posq[:, 0:1]),
                       jnp.where(axis_y, posk[:, 1:2], posk[:, 0:1]))
    head_v = jnp.where(sec < 10, hq, hk)
    val = jnp.where(sec < 8, rope_v * fr, head_v)
    hi = (sec >> 3) & 1
    sin_i = ((sec >> 1) & 1) * (1 - hi) + (sec & 1) * hi
    ang = val - sin_i.astype(jnp.float32) * jnp.float32(jnp.pi / 2)
    C = jnp.cos(ang)
    q16 = quarter
    return (C[:, 0:2 * q16], C[:, 2 * q16:4 * q16],
            C[:, 4 * q16:6 * q16], C[:, 6 * q16:8 * q16],
            C[:, 8 * q16:8 * q16 + 1], C[:, 9 * q16:9 * q16 + 1],
            C[:, 10 * q16:10 * q16 + 1], C[:, 11 * q16:11 * q16 + 1])


def _encode_heads(P, rc, rs, hc, hs, H, DH, gain):
    half = DH // 2
    heads = []
    for h in range(H):
        p1 = P[:, h * DH:h * DH + half]
        p2 = P[:, h * DH + half:(h + 1) * DH]
        c, s = (rc, rs) if h % 2 == 0 else (hc, hs)
        heads.append(jnp.concatenate(
            [(p1 * c - p2 * s) * gain, (p2 * c + p1 * s) * gain],
            axis=1).astype(jnp.bfloat16))
    return heads


def _proj_encode_kernel(qf, kvf, posq, posk, hq, hk,
                        wq, bq, wk, bk, wv, bv,
                        qe, ke, vv, *, H, DH):
    dn = (((1,), (1,)), ((), ()))
    quarter = DH // 4
    scale = 1.0 / (DH ** 0.5)
    Pq = jax.lax.dot_general(qf[0], wq[...], dn,
                             preferred_element_type=jnp.float32) + bq[...]
    Pk = jax.lax.dot_general(kvf[0], wk[...], dn,
                             preferred_element_type=jnp.float32) + bk[...]
    Pv = jax.lax.dot_general(kvf[0], wv[...], dn,
                             preferred_element_type=jnp.float32) + bv[...]

    rcq, rsq, rck, rsk, hcq, hsq, hck, hsk = _rot_tables_qk(
        posq[0], posk[0], hq[0], hk[0], quarter)
    for h, piece in enumerate(_encode_heads(Pq, rcq, rsq, hcq, hsq, H, DH, scale)):
        qe[0, h] = piece
    for h, piece in enumerate(_encode_heads(Pk, rck, rsk, hck, hsk, H, DH, 1.0)):
        ke[0, h] = piece
    for h in range(H):
        vv[0, h] = Pv[:, h * DH:(h + 1) * DH].astype(jnp.bfloat16)


def _attn_kernel(qe, ke, vv, posq, poskT, wo, bo, out, *, H, DH, K, NK):
    BQ = out.shape[1]

    xq = posq[0][:, 0:1]
    yq = posq[0][:, 1:2]
    xk = poskT[0][0:1, :]
    yk = poskT[0][1:2, :]
    d2 = (xq - xk) ** 2 + (yq - yk) ** 2

    r = jax.lax.bitcast_convert_type(d2, jnp.int32)

    ones_col = jnp.ones((128, 1), jnp.float32)
    cdn = (((1,), (0,)), ((), ()))
    v32 = jnp.zeros((BQ, 1), jnp.int32)
    for bit in range(30, -1, -1):
        cand = v32 | jnp.int32(1 << bit)
        m = jnp.where(r < cand, jnp.float32(1), jnp.float32(0))
        part = ((m[:, 0:128] + m[:, 128:256]) + (m[:, 256:384] + m[:, 384:512])) + \
               ((m[:, 512:640] + m[:, 640:768]) + (m[:, 768:896] + m[:, 896:1024]))
        cnt = jax.lax.dot_general(part, ones_col, cdn,
                                  preferred_element_type=jnp.float32)
        v32 = jnp.where(cnt <= jnp.float32(K - 1), cand, v32)
    keep = r <= v32

    dn = (((1,), (1,)), ((), ()))
    outs = []
    for h in range(H):
        lg = jax.lax.dot_general(qe[0, h], ke[0, h], dn,
                                 preferred_element_type=jnp.float32)
        p = jnp.where(keep, jnp.exp(lg), 0.0)
        inv = 1.0 / jnp.sum(p, axis=1, keepdims=True)
        outs.append(jnp.dot(p.astype(jnp.bfloat16), vv[0, h],
                            preferred_element_type=jnp.float32) * inv)
    O = jnp.concatenate(outs, axis=1)
    out[0] = jax.lax.dot_general(O, wo[...], dn,
                                 preferred_element_type=jnp.float32) + bo[...]


def kernel(q_feat, kv_feat, pos_q, pos_k, heading_q, heading_k, mask_k,
           Wq, bq, Wk, bk, Wv, bv, Wo, bo):
    B, NQ, D = q_feat.shape
    NK = kv_feat.shape[1]
    H = 8
    DH = D // H
    K = 32
    BQ = 512
    nq_blocks = NQ // BQ
    BQA = 512
    na_blocks = NQ // BQA

    poskT = jnp.swapaxes(pos_k, 1, 2)
    hq3 = heading_q[..., None]
    hk3 = heading_k[..., None]
    bq2 = bq[None, :]
    bk2 = bk[None, :]
    bv2 = bv[None, :]
    bo2 = bo[None, :]

    enc_shape = jax.ShapeDtypeStruct((B, H, NQ, DH), jnp.bfloat16)
    val_shape = jax.ShapeDtypeStruct((B, H, NQ, DH), jnp.bfloat16)
    qe, ke, vv = pl.pallas_call(
        functools.partial(_proj_encode_kernel, H=H, DH=DH),
        grid=(B, na_blocks),
        in_specs=[
            pl.BlockSpec((1, BQA, D), lambda b, i: (b, i, 0)),
            pl.BlockSpec((1, BQA, D), lambda b, i: (b, i, 0)),
            pl.BlockSpec((1, BQA, 2), lambda b, i: (b, i, 0)),
            pl.BlockSpec((1, BQA, 2), lambda b, i: (b, i, 0)),
            pl.BlockSpec((1, BQA, 1), lambda b, i: (b, i, 0)),
            pl.BlockSpec((1, BQA, 1), lambda b, i: (b, i, 0)),
            pl.BlockSpec((D, D), lambda b, i: (0, 0)),
            pl.BlockSpec((1, D), lambda b, i: (0, 0)),
            pl.BlockSpec((D, D), lambda b, i: (0, 0)),
            pl.BlockSpec((1, D), lambda b, i: (0, 0)),
            pl.BlockSpec((D, D), lambda b, i: (0, 0)),
            pl.BlockSpec((1, D), lambda b, i: (0, 0)),
        ],
        out_specs=[
            pl.BlockSpec((1, H, BQA, DH), lambda b, i: (b, 0, i, 0)),
            pl.BlockSpec((1, H, BQA, DH), lambda b, i: (b, 0, i, 0)),
            pl.BlockSpec((1, H, BQA, DH), lambda b, i: (b, 0, i, 0)),
        ],
        out_shape=[enc_shape, enc_shape, val_shape],
        compiler_params=pltpu.CompilerParams(
            dimension_semantics=("parallel", "parallel")),
    )(q_feat, kv_feat, pos_q, pos_k, hq3, hk3, Wq, bq2, Wk, bk2, Wv, bv2)

    out = pl.pallas_call(
        functools.partial(_attn_kernel, H=H, DH=DH, K=K, NK=NK),
        grid=(B, nq_blocks),
        in_specs=[
            pl.BlockSpec((1, H, BQ, DH), lambda b, i: (b, 0, i, 0)),
            pl.BlockSpec((1, H, NK, DH), lambda b, i: (b, 0, 0, 0)),
            pl.BlockSpec((1, H, NK, DH), lambda b, i: (b, 0, 0, 0)),
            pl.BlockSpec((1, BQ, 2), lambda b, i: (b, i, 0)),
            pl.BlockSpec((1, 2, NK), lambda b, i: (b, 0, 0)),
            pl.BlockSpec((D, D), lambda b, i: (0, 0)),
            pl.BlockSpec((1, D), lambda b, i: (0, 0)),
        ],
        out_specs=pl.BlockSpec((1, BQ, D), lambda b, i: (b, i, 0)),
        out_shape=jax.ShapeDtypeStruct((B, NQ, D), jnp.float32),
        compiler_params=pltpu.CompilerParams(
            dimension_semantics=("parallel", "parallel")),
    )(qe, ke, vv, pos_q, poskT, Wo, bo2)

    return out
